# baseline (speedup 1.0000x reference)
import functools

import jax
import jax.numpy as jnp
from jax import lax
from jax.experimental import pallas as pl
from jax.experimental.pallas import tpu as pltpu
from jax.experimental.pallas import tpu_sc as plsc

N = 10000
E = 320000
D = 128

NC = 2
NS = 16
NW = NC * NS
EPW = E // NW
CH = 40
NCH = EPW // CH
NP = 10240
RPS = NP // NS
ZR = 16

_mesh = plsc.VectorSubcoreMesh(core_axis_name="c", subcore_axis_name="s")


def _zero_2d(ref, rows, cols):
    zero = jnp.zeros((16,), jnp.float32)
    cchunks = cols // 16

    def body(i, carry):
        r = i // cchunks
        j = i % cchunks
        ref[r, pl.ds(j * 16, 16)] = zero
        return carry

    lax.fori_loop(0, rows * cchunks, body, 0)


K = 5
K2 = 2 * K
STEADY = NCH - K2


@functools.partial(
    pl.kernel,
    out_type=jax.ShapeDtypeStruct((NC, NP, D), jnp.float32),
    mesh=_mesh,
    scratch_types=[
        pltpu.VMEM_SHARED((NP, D), jnp.float32),
        pltpu.VMEM((K2, CH), jnp.int32),
        pltpu.VMEM((K2, CH), jnp.int32),
        pltpu.VMEM((K, CH, D), jnp.float32),
        pltpu.VMEM((ZR, D), jnp.float32),
        [pltpu.SemaphoreType.DMA] * K2,
        [pltpu.SemaphoreType.DMA] * K,
        [pltpu.SemaphoreType.DMA] * K,
        pltpu.SemaphoreType.DMA,
    ],
)
def _sc_segsum(h_hbm, src_hbm, dst_hbm, out_hbm, acc_sh, srcr, dstr,
               rows_v, stage_v, semi, semg, sems, semd):
    c = lax.axis_index("c")
    s = lax.axis_index("s")
    wid = s * NC + c
    row0 = s * RPS

    def fire_idx(g, slot):
        pltpu.async_copy(src_hbm.at[wid, g], srcr.at[slot], semi[slot])
        pltpu.async_copy(dst_hbm.at[wid, g], dstr.at[slot], semi[slot])

    def wait_idx(slot):
        pltpu.make_async_copy(src_hbm.at[wid, 0], srcr.at[slot],
                              semi[slot]).wait()
        pltpu.make_async_copy(dst_hbm.at[wid, 0], dstr.at[slot],
                              semi[slot]).wait()

    def fire_gather(slot, rslot):
        pltpu.async_copy(h_hbm.at[srcr.at[slot]], rows_v.at[rslot],
                         semg[rslot])

    def wait_gather(slot, rslot):
        pltpu.make_async_copy(h_hbm.at[srcr.at[slot]], rows_v.at[rslot],
                              semg[rslot]).wait()

    def fire_scatter(slot, rslot):
        pltpu.async_copy(rows_v.at[rslot], acc_sh.at[dstr.at[slot]],
                         sems[rslot], add=True)

    def wait_scatter(slot, rslot):
        pltpu.make_async_copy(rows_v.at[rslot], acc_sh.at[dstr.at[slot]],
                              sems[rslot]).wait()

    for g in range(K2):
        fire_idx(g, g)

    _zero_2d(stage_v, ZR, D)
    for j in range(RPS // ZR):
        pltpu.async_copy(stage_v, acc_sh.at[pl.ds(row0 + j * ZR, ZR)], semd)
    for j in range(RPS // ZR):
        pltpu.make_async_copy(stage_v, acc_sh.at[pl.ds(row0, ZR)], semd).wait()
    plsc.subcore_barrier()

    for g in range(K):
        wait_idx(g)
        fire_gather(g, g)

    def outer(o, carry):
        for b in range(K2):
            g = o * K2 + b
            rb = b % K
            wait_gather(b, rb)
            fire_scatter(b, rb)
            wait_scatter(b, rb)
            fire_idx(g + K2, b)
            wait_idx((b + K) % K2)
            fire_gather((b + K) % K2, rb)
        return carry

    lax.fori_loop(0, STEADY // K2, outer, 0)

    for g in range(STEADY, NCH):
        b = g % K2
        rb = b % K
        wait_gather(b, rb)
        fire_scatter(b, rb)
        wait_scatter(b, rb)
        if g + K < NCH:
            wait_idx((b + K) % K2)
            fire_gather((b + K) % K2, rb)

    plsc.subcore_barrier()

    for j in range(RPS // ZR):
        r = row0 + j * ZR
        pltpu.async_copy(acc_sh.at[pl.ds(r, ZR)], out_hbm.at[c, pl.ds(r, ZR)],
                         semd)
    for j in range(RPS // ZR):
        pltpu.make_async_copy(acc_sh.at[pl.ds(row0, ZR)],
                              out_hbm.at[c, pl.ds(row0, ZR)], semd).wait()


@functools.partial(
    pl.kernel,
    out_type=jax.ShapeDtypeStruct((NC, NP, D), jnp.float32),
    mesh=_mesh,
    scratch_types=[
        pltpu.VMEM_SHARED((NP, D), jnp.float32),
        pltpu.VMEM((NCH, CH), jnp.int32),
        pltpu.VMEM((CH, D), jnp.float32),
        pltpu.VMEM((ZR, D), jnp.float32),
        [pltpu.SemaphoreType.DMA] * K,
        pltpu.SemaphoreType.DMA,
    ],
)
def _sc_count(dst_hbm, out_hbm, cnt_sh, dst_v, ones_v, stage_v, sems, semd):
    c = lax.axis_index("c")
    s = lax.axis_index("s")
    wid = s * NC + c
    row0 = s * RPS

    pltpu.sync_copy(dst_hbm.at[wid], dst_v)

    _zero_2d(stage_v, ZR, D)
    for j in range(RPS // ZR):
        pltpu.async_copy(stage_v, cnt_sh.at[pl.ds(row0 + j * ZR, ZR)], semd)

    one = jnp.ones((16,), jnp.float32)

    def fill(i, carry):
        r = i // 8
        j = i % 8
        ones_v[r, pl.ds(j * 16, 16)] = one
        return carry

    lax.fori_loop(0, CH * 8, fill, 0)
    for j in range(RPS // ZR):
        pltpu.make_async_copy(stage_v, cnt_sh.at[pl.ds(row0, ZR)], semd).wait()
    plsc.subcore_barrier()

    for b in range(K):
        pltpu.async_copy(ones_v, cnt_sh.at[dst_v.at[b]], sems[b], add=True)

    def outer(o, carry):
        for b in range(K):
            g = o * K + b
            pltpu.make_async_copy(
                ones_v, cnt_sh.at[dst_v.at[g]], sems[b]).wait()

            @pl.when(g + K < NCH)
            def _():
                pltpu.async_copy(
                    ones_v, cnt_sh.at[dst_v.at[g + K]], sems[b], add=True)
        return carry

    lax.fori_loop(0, NCH // K, outer, 0)
    plsc.subcore_barrier()

    for j in range(RPS // ZR):
        r = row0 + j * ZR
        pltpu.async_copy(cnt_sh.at[pl.ds(r, ZR)], out_hbm.at[c, pl.ds(r, ZR)],
                         semd)
    for j in range(RPS // ZR):
        pltpu.make_async_copy(cnt_sh.at[pl.ds(row0, ZR)],
                              out_hbm.at[c, pl.ds(row0, ZR)], semd).wait()


def _tc_layer_body(with_residual, h_ref, a0_ref, a1_ref, c0_ref, c1_ref,
                   wl_ref, bl_ref, wr_ref, lnw_ref, lnb_ref, pw_ref, out_ref):
    cnt = c0_ref[:, 0:1] + c1_ref[:, 0:1]
    mean = (a0_ref[...] + a1_ref[...]) / jnp.maximum(cnt, 1.0)
    t = (
        jnp.dot(mean, wl_ref[...], preferred_element_type=jnp.float32,
                precision=lax.Precision.HIGHEST)
        + jnp.dot(h_ref[...], wr_ref[...], preferred_element_type=jnp.float32,
                  precision=lax.Precision.HIGHEST)
        + bl_ref[...]
    )
    if with_residual:
        t = t + h_ref[...]
    mu = jnp.mean(t, axis=-1, keepdims=True)
    var = jnp.mean((t - mu) ** 2, axis=-1, keepdims=True)
    t = (t - mu) * lax.rsqrt(var + 1e-5) * lnw_ref[...] + lnb_ref[...]
    w = pw_ref[0, 0]
    out_ref[...] = jnp.where(t > 0, t, w * t)


def _tc_layer(h, a0, a1, c0, c1, WlT, bl, WrT, lnw, lnb, pw, with_residual):
    R = 1000
    grid = (N // R,)
    row_spec = pl.BlockSpec((R, D), lambda i: (i, 0))
    cnt_spec = pl.BlockSpec((R, 16), lambda i: (i, 0))

    def full(shape):
        return pl.BlockSpec(shape, lambda i: (0,) * len(shape))

    return pl.pallas_call(
        functools.partial(_tc_layer_body, with_residual),
        grid=grid,
        in_specs=[
            row_spec, row_spec, row_spec, cnt_spec, cnt_spec,
            full((D, D)), full((1, D)), full((D, D)),
            full((1, D)), full((1, D)), full((1, 1)),
        ],
        out_specs=row_spec,
        out_shape=jax.ShapeDtypeStruct((N, D), jnp.float32),
    )(h, a0, a1, c0, c1, WlT, bl, WrT, lnw, lnb, pw)


def kernel(x, edge_index, Wl0, bl0, Wr0, lnw0, lnb0, Wl1, bl1, Wr1, lnw1,
           lnb1, Wl2, bl2, Wr2, lnw2, lnb2, prelu_w):
    ei = edge_index.astype(jnp.int32)
    src = ei[0].reshape(NW, NCH, CH)
    dst = ei[1].reshape(NW, NCH, CH)

    cnt = _sc_count(dst)[:, :N, :16]
    c0, c1 = cnt[0], cnt[1]
    pw = prelu_w.reshape(1, 1)

    params = [
        (Wl0, bl0, Wr0, lnw0, lnb0),
        (Wl1, bl1, Wr1, lnw1, lnb1),
        (Wl2, bl2, Wr2, lnw2, lnb2),
    ]
    h = x
    for i, (Wl, bl, Wr, lnw, lnb) in enumerate(params):
        agg = _sc_segsum(h, src, dst)[:, :N]
        h = _tc_layer(h, agg[0], agg[1], c0, c1, Wl.T, bl.reshape(1, D),
                      Wr.T, lnw.reshape(1, D), lnb.reshape(1, D), pw, i > 0)
    return h


# speedup vs baseline: 1.0273x; 1.0273x over previous
import functools

import jax
import jax.numpy as jnp
from jax import lax
from jax.experimental import pallas as pl
from jax.experimental.pallas import tpu as pltpu
from jax.experimental.pallas import tpu_sc as plsc

N = 10000
E = 320000
D = 128

NC = 2
NS = 16
NW = NC * NS
EPW = E // NW
CH = 40
NCH = EPW // CH
NP = 10240
RPS = NP // NS
ZR = 16

_mesh = plsc.VectorSubcoreMesh(core_axis_name="c", subcore_axis_name="s")


def _zero_2d(ref, rows, cols):
    zero = jnp.zeros((16,), jnp.float32)
    cchunks = cols // 16

    def body(i, carry):
        r = i // cchunks
        j = i % cchunks
        ref[r, pl.ds(j * 16, 16)] = zero
        return carry

    lax.fori_loop(0, rows * cchunks, body, 0)


K = 5
K2 = 2 * K
STEADY = NCH - K2


@functools.partial(
    pl.kernel,
    out_type=jax.ShapeDtypeStruct((NC, NP, D), jnp.float32),
    mesh=_mesh,
    scratch_types=[
        pltpu.VMEM_SHARED((NP, D), jnp.float32),
        pltpu.VMEM((K2, CH), jnp.int32),
        pltpu.VMEM((K2, CH), jnp.int32),
        pltpu.VMEM((K, CH, D), jnp.float32),
        pltpu.VMEM((ZR, D), jnp.float32),
        [pltpu.SemaphoreType.DMA] * K2,
        [pltpu.SemaphoreType.DMA] * K,
        [pltpu.SemaphoreType.DMA] * K,
        pltpu.SemaphoreType.DMA,
    ],
)
def _sc_segsum(h_hbm, src_hbm, dst_hbm, out_hbm, acc_sh, srcr, dstr,
               rows_v, stage_v, semi, semg, sems, semd):
    c = lax.axis_index("c")
    s = lax.axis_index("s")
    wid = s * NC + c
    row0 = s * RPS

    def fire_idx(g, slot):
        pltpu.async_copy(src_hbm.at[wid, g], srcr.at[slot], semi[slot])
        pltpu.async_copy(dst_hbm.at[wid, g], dstr.at[slot], semi[slot])

    def wait_idx(slot):
        pltpu.make_async_copy(src_hbm.at[wid, 0], srcr.at[slot],
                              semi[slot]).wait()
        pltpu.make_async_copy(dst_hbm.at[wid, 0], dstr.at[slot],
                              semi[slot]).wait()

    def fire_gather(slot, rslot):
        pltpu.async_copy(h_hbm.at[srcr.at[slot]], rows_v.at[rslot],
                         semg[rslot])

    def wait_gather(slot, rslot):
        pltpu.make_async_copy(h_hbm.at[srcr.at[slot]], rows_v.at[rslot],
                              semg[rslot]).wait()

    def fire_scatter(slot, rslot):
        pltpu.async_copy(rows_v.at[rslot], acc_sh.at[dstr.at[slot]],
                         sems[rslot], add=True)

    def wait_scatter(slot, rslot):
        pltpu.make_async_copy(rows_v.at[rslot], acc_sh.at[dstr.at[slot]],
                              sems[rslot]).wait()

    for g in range(K2):
        fire_idx(g, g)

    _zero_2d(stage_v, ZR, D)
    for j in range(RPS // ZR):
        pltpu.async_copy(stage_v, acc_sh.at[pl.ds(row0 + j * ZR, ZR)], semd)
    for j in range(RPS // ZR):
        pltpu.make_async_copy(stage_v, acc_sh.at[pl.ds(row0, ZR)], semd).wait()
    plsc.subcore_barrier()

    for g in range(K):
        wait_idx(g)
        fire_gather(g, g)

    def outer(o, carry):
        for b in range(K2):
            g = o * K2 + b
            rb = b % K
            wait_gather(b, rb)
            fire_scatter(b, rb)
            wait_scatter(b, rb)
            fire_idx(g + K2, b)
            wait_idx((b + K) % K2)
            fire_gather((b + K) % K2, rb)
        return carry

    lax.fori_loop(0, STEADY // K2, outer, 0)

    for g in range(STEADY, NCH):
        b = g % K2
        rb = b % K
        wait_gather(b, rb)
        fire_scatter(b, rb)
        wait_scatter(b, rb)
        if g + K < NCH:
            wait_idx((b + K) % K2)
            fire_gather((b + K) % K2, rb)

    plsc.subcore_barrier()

    for j in range(RPS // ZR):
        r = row0 + j * ZR
        pltpu.async_copy(acc_sh.at[pl.ds(r, ZR)], out_hbm.at[c, pl.ds(r, ZR)],
                         semd)
    for j in range(RPS // ZR):
        pltpu.make_async_copy(acc_sh.at[pl.ds(row0, ZR)],
                              out_hbm.at[c, pl.ds(row0, ZR)], semd).wait()


@functools.partial(
    pl.kernel,
    out_type=jax.ShapeDtypeStruct((NC, NP, D), jnp.float32),
    mesh=_mesh,
    scratch_types=[
        pltpu.VMEM_SHARED((NP, D), jnp.float32),
        pltpu.VMEM((NCH, CH), jnp.int32),
        pltpu.VMEM((CH, D), jnp.float32),
        pltpu.VMEM((ZR, D), jnp.float32),
        [pltpu.SemaphoreType.DMA] * K,
        pltpu.SemaphoreType.DMA,
    ],
)
def _sc_count(dst_hbm, out_hbm, cnt_sh, dst_v, ones_v, stage_v, sems, semd):
    c = lax.axis_index("c")
    s = lax.axis_index("s")
    wid = s * NC + c
    row0 = s * RPS

    pltpu.sync_copy(dst_hbm.at[wid], dst_v)

    _zero_2d(stage_v, ZR, D)
    for j in range(RPS // ZR):
        pltpu.async_copy(stage_v, cnt_sh.at[pl.ds(row0 + j * ZR, ZR)], semd)

    one = jnp.ones((16,), jnp.float32)

    def fill(i, carry):
        r = i // 8
        j = i % 8
        ones_v[r, pl.ds(j * 16, 16)] = one
        return carry

    lax.fori_loop(0, CH * 8, fill, 0)
    for j in range(RPS // ZR):
        pltpu.make_async_copy(stage_v, cnt_sh.at[pl.ds(row0, ZR)], semd).wait()
    plsc.subcore_barrier()

    for b in range(K):
        pltpu.async_copy(ones_v, cnt_sh.at[dst_v.at[b]], sems[b], add=True)

    def outer(o, carry):
        for b in range(K):
            g = o * K + b
            pltpu.make_async_copy(
                ones_v, cnt_sh.at[dst_v.at[g]], sems[b]).wait()

            @pl.when(g + K < NCH)
            def _():
                pltpu.async_copy(
                    ones_v, cnt_sh.at[dst_v.at[g + K]], sems[b], add=True)
        return carry

    lax.fori_loop(0, NCH // K, outer, 0)
    plsc.subcore_barrier()

    for j in range(RPS // ZR):
        r = row0 + j * ZR
        pltpu.async_copy(cnt_sh.at[pl.ds(r, ZR)], out_hbm.at[c, pl.ds(r, ZR)],
                         semd)
    for j in range(RPS // ZR):
        pltpu.make_async_copy(cnt_sh.at[pl.ds(row0, ZR)],
                              out_hbm.at[c, pl.ds(row0, ZR)], semd).wait()


_R = 1000
_row_spec = pl.BlockSpec((_R, D), lambda i: (i, 0))
_cnt_spec = pl.BlockSpec((_R, 16), lambda i: (i, 0))


def _full(shape):
    return pl.BlockSpec(shape, lambda i: (0,) * len(shape))


def _tc_pre_body(with_residual, h_ref, wr_ref, bl_ref, out_ref):
    t = jnp.dot(h_ref[...], wr_ref[...], preferred_element_type=jnp.float32,
                precision=lax.Precision.HIGHEST) + bl_ref[...]
    if with_residual:
        t = t + h_ref[...]
    out_ref[...] = t


def _tc_pre(h, WrT, bl, with_residual):
    return pl.pallas_call(
        functools.partial(_tc_pre_body, with_residual),
        grid=(N // _R,),
        in_specs=[_row_spec, _full((D, D)), _full((1, D))],
        out_specs=_row_spec,
        out_shape=jax.ShapeDtypeStruct((N, D), jnp.float32),
    )(h, WrT, bl)


def _tc_post_body(a0_ref, a1_ref, c0_ref, c1_ref, wl_ref, tmp_ref,
                  lnw_ref, lnb_ref, pw_ref, out_ref):
    cnt = c0_ref[:, 0:1] + c1_ref[:, 0:1]
    mean = (a0_ref[...] + a1_ref[...]) / jnp.maximum(cnt, 1.0)
    t = jnp.dot(mean, wl_ref[...], preferred_element_type=jnp.float32,
                precision=lax.Precision.HIGHEST) + tmp_ref[...]
    mu = jnp.mean(t, axis=-1, keepdims=True)
    var = jnp.mean((t - mu) ** 2, axis=-1, keepdims=True)
    t = (t - mu) * lax.rsqrt(var + 1e-5) * lnw_ref[...] + lnb_ref[...]
    w = pw_ref[0, 0]
    out_ref[...] = jnp.where(t > 0, t, w * t)


def _tc_post(a0, a1, c0, c1, WlT, tmp, lnw, lnb, pw):
    return pl.pallas_call(
        _tc_post_body,
        grid=(N // _R,),
        in_specs=[
            _row_spec, _row_spec, _cnt_spec, _cnt_spec,
            _full((D, D)), _row_spec, _full((1, D)), _full((1, D)),
            _full((1, 1)),
        ],
        out_specs=_row_spec,
        out_shape=jax.ShapeDtypeStruct((N, D), jnp.float32),
    )(a0, a1, c0, c1, WlT, tmp, lnw, lnb, pw)


def kernel(x, edge_index, Wl0, bl0, Wr0, lnw0, lnb0, Wl1, bl1, Wr1, lnw1,
           lnb1, Wl2, bl2, Wr2, lnw2, lnb2, prelu_w):
    ei = edge_index.astype(jnp.int32)
    src = ei[0].reshape(NW, NCH, CH)
    dst = ei[1].reshape(NW, NCH, CH)

    cnt = _sc_count(dst)[:, :N, :16]
    c0, c1 = cnt[0], cnt[1]
    pw = prelu_w.reshape(1, 1)

    params = [
        (Wl0, bl0, Wr0, lnw0, lnb0),
        (Wl1, bl1, Wr1, lnw1, lnb1),
        (Wl2, bl2, Wr2, lnw2, lnb2),
    ]
    h = x
    for i, (Wl, bl, Wr, lnw, lnb) in enumerate(params):
        agg = _sc_segsum(h, src, dst)[:, :N]
        tmp = _tc_pre(h, Wr.T, bl.reshape(1, D), i > 0)
        h = _tc_post(agg[0], agg[1], c0, c1, Wl.T, tmp,
                     lnw.reshape(1, D), lnb.reshape(1, D), pw)
    return h


# speedup vs baseline: 1.0651x; 1.0368x over previous
import functools

import jax
import jax.numpy as jnp
from jax import lax
from jax.experimental import pallas as pl
from jax.experimental.pallas import tpu as pltpu
from jax.experimental.pallas import tpu_sc as plsc

N = 10000
E = 320000
D = 128

NC = 2
NS = 16
NW = NC * NS
EPW = E // NW
CH = 40
NCH = EPW // CH
NP = 10240
RPS = NP // NS
ZR = 16

_mesh = plsc.VectorSubcoreMesh(core_axis_name="c", subcore_axis_name="s")


def _zero_2d(ref, rows, cols):
    zero = jnp.zeros((16,), jnp.float32)
    cchunks = cols // 16

    def body(i, carry):
        r = i // cchunks
        j = i % cchunks
        ref[r, pl.ds(j * 16, 16)] = zero
        return carry

    lax.fori_loop(0, rows * cchunks, body, 0)


K = 5
K2 = 2 * K
STEADY = NCH - K2


@functools.partial(
    pl.kernel,
    out_type=jax.ShapeDtypeStruct((NC, NP, D), jnp.float32),
    mesh=_mesh,
    scratch_types=[
        pltpu.VMEM_SHARED((NP, D), jnp.float32),
        pltpu.VMEM((K2, CH), jnp.int32),
        pltpu.VMEM((K2, CH), jnp.int32),
        pltpu.VMEM((K, CH, D), jnp.float32),
        pltpu.VMEM((ZR, D), jnp.float32),
        [pltpu.SemaphoreType.DMA] * K2,
        [pltpu.SemaphoreType.DMA] * K,
        [pltpu.SemaphoreType.DMA] * K,
        pltpu.SemaphoreType.DMA,
    ],
)
def _sc_segsum(h_hbm, src_hbm, dst_hbm, out_hbm, acc_sh, srcr, dstr,
               rows_v, stage_v, semi, semg, sems, semd):
    c = lax.axis_index("c")
    s = lax.axis_index("s")
    wid = s * NC + c
    row0 = s * RPS

    def fire_idx(g, slot):
        pltpu.async_copy(src_hbm.at[wid, g], srcr.at[slot], semi[slot])
        pltpu.async_copy(dst_hbm.at[wid, g], dstr.at[slot], semi[slot])

    def wait_idx(slot):
        pltpu.make_async_copy(src_hbm.at[wid, 0], srcr.at[slot],
                              semi[slot]).wait()
        pltpu.make_async_copy(dst_hbm.at[wid, 0], dstr.at[slot],
                              semi[slot]).wait()

    def fire_gather(slot, rslot):
        pltpu.async_copy(h_hbm.at[srcr.at[slot]], rows_v.at[rslot],
                         semg[rslot])

    def wait_gather(slot, rslot):
        pltpu.make_async_copy(h_hbm.at[srcr.at[slot]], rows_v.at[rslot],
                              semg[rslot]).wait()

    def fire_scatter(slot, rslot):
        pltpu.async_copy(rows_v.at[rslot], acc_sh.at[dstr.at[slot]],
                         sems[rslot], add=True)

    def wait_scatter(slot, rslot):
        pltpu.make_async_copy(rows_v.at[rslot], acc_sh.at[dstr.at[slot]],
                              sems[rslot]).wait()

    for g in range(K2):
        fire_idx(g, g)

    _zero_2d(stage_v, ZR, D)
    for j in range(RPS // ZR):
        pltpu.async_copy(stage_v, acc_sh.at[pl.ds(row0 + j * ZR, ZR)], semd)
    for j in range(RPS // ZR):
        pltpu.make_async_copy(stage_v, acc_sh.at[pl.ds(row0, ZR)], semd).wait()
    plsc.subcore_barrier()

    for g in range(K):
        wait_idx(g)
        fire_gather(g, g)

    def outer(o, carry):
        for b in range(K2):
            g = o * K2 + b
            rb = b % K
            wait_gather(b, rb)
            fire_scatter(b, rb)
            wait_scatter(b, rb)
            fire_idx(g + K2, b)
            wait_idx((b + K) % K2)
            fire_gather((b + K) % K2, rb)
        return carry

    lax.fori_loop(0, STEADY // K2, outer, 0)

    for g in range(STEADY, NCH):
        b = g % K2
        rb = b % K
        wait_gather(b, rb)
        fire_scatter(b, rb)
        wait_scatter(b, rb)
        if g + K < NCH:
            wait_idx((b + K) % K2)
            fire_gather((b + K) % K2, rb)

    plsc.subcore_barrier()

    for j in range(RPS // ZR):
        r = row0 + j * ZR
        pltpu.async_copy(acc_sh.at[pl.ds(r, ZR)], out_hbm.at[c, pl.ds(r, ZR)],
                         semd)
    for j in range(RPS // ZR):
        pltpu.make_async_copy(acc_sh.at[pl.ds(row0, ZR)],
                              out_hbm.at[c, pl.ds(row0, ZR)], semd).wait()


@functools.partial(
    pl.kernel,
    out_type=jax.ShapeDtypeStruct((NC, NP, D), jnp.float32),
    mesh=_mesh,
    scratch_types=[
        pltpu.VMEM_SHARED((NP, D), jnp.float32),
        pltpu.VMEM((NCH, CH), jnp.int32),
        pltpu.VMEM((CH, D), jnp.float32),
        pltpu.VMEM((ZR, D), jnp.float32),
        [pltpu.SemaphoreType.DMA] * K,
        pltpu.SemaphoreType.DMA,
    ],
)
def _sc_count(dst_hbm, out_hbm, cnt_sh, dst_v, ones_v, stage_v, sems, semd):
    c = lax.axis_index("c")
    s = lax.axis_index("s")
    wid = s * NC + c
    row0 = s * RPS

    pltpu.sync_copy(dst_hbm.at[wid], dst_v)

    _zero_2d(stage_v, ZR, D)
    for j in range(RPS // ZR):
        pltpu.async_copy(stage_v, cnt_sh.at[pl.ds(row0 + j * ZR, ZR)], semd)

    one = jnp.ones((16,), jnp.float32)

    def fill(i, carry):
        r = i // 8
        j = i % 8
        ones_v[r, pl.ds(j * 16, 16)] = one
        return carry

    lax.fori_loop(0, CH * 8, fill, 0)
    for j in range(RPS // ZR):
        pltpu.make_async_copy(stage_v, cnt_sh.at[pl.ds(row0, ZR)], semd).wait()
    plsc.subcore_barrier()

    for b in range(K):
        pltpu.async_copy(ones_v, cnt_sh.at[dst_v.at[b]], sems[b], add=True)

    def outer(o, carry):
        for b in range(K):
            g = o * K + b
            pltpu.make_async_copy(
                ones_v, cnt_sh.at[dst_v.at[g]], sems[b]).wait()

            @pl.when(g + K < NCH)
            def _():
                pltpu.async_copy(
                    ones_v, cnt_sh.at[dst_v.at[g + K]], sems[b], add=True)
        return carry

    lax.fori_loop(0, NCH // K, outer, 0)
    plsc.subcore_barrier()

    for j in range(RPS // ZR):
        r = row0 + j * ZR
        pltpu.async_copy(cnt_sh.at[pl.ds(r, ZR)], out_hbm.at[c, pl.ds(r, ZR)],
                         semd)
    for j in range(RPS // ZR):
        pltpu.make_async_copy(cnt_sh.at[pl.ds(row0, ZR)],
                              out_hbm.at[c, pl.ds(row0, ZR)], semd).wait()


_R = 2000
_row_spec = pl.BlockSpec((_R, D), lambda i: (i, 0))
_cnt_spec = pl.BlockSpec((_R, 16), lambda i: (i, 0))


def _full(shape):
    return pl.BlockSpec(shape, lambda i: (0,) * len(shape))


def _tc_pre_body(with_residual, h_ref, wr_ref, bl_ref, out_ref):
    t = jnp.dot(h_ref[...], wr_ref[...], preferred_element_type=jnp.float32,
                precision=lax.Precision.DEFAULT) + bl_ref[...]
    if with_residual:
        t = t + h_ref[...]
    out_ref[...] = t


def _tc_pre(h, WrT, bl, with_residual):
    return pl.pallas_call(
        functools.partial(_tc_pre_body, with_residual),
        grid=(N // _R,),
        in_specs=[_row_spec, _full((D, D)), _full((1, D))],
        out_specs=_row_spec,
        out_shape=jax.ShapeDtypeStruct((N, D), jnp.float32),
    )(h, WrT, bl)


def _tc_post_body(a0_ref, a1_ref, c0_ref, c1_ref, wl_ref, tmp_ref,
                  lnw_ref, lnb_ref, pw_ref, out_ref):
    cnt = c0_ref[:, 0:1] + c1_ref[:, 0:1]
    mean = (a0_ref[...] + a1_ref[...]) / jnp.maximum(cnt, 1.0)
    t = jnp.dot(mean, wl_ref[...], preferred_element_type=jnp.float32,
                precision=lax.Precision.DEFAULT) + tmp_ref[...]
    mu = jnp.mean(t, axis=-1, keepdims=True)
    var = jnp.mean((t - mu) ** 2, axis=-1, keepdims=True)
    t = (t - mu) * lax.rsqrt(var + 1e-5) * lnw_ref[...] + lnb_ref[...]
    w = pw_ref[0, 0]
    out_ref[...] = jnp.where(t > 0, t, w * t)


def _tc_post(a0, a1, c0, c1, WlT, tmp, lnw, lnb, pw):
    return pl.pallas_call(
        _tc_post_body,
        grid=(N // _R,),
        in_specs=[
            _row_spec, _row_spec, _cnt_spec, _cnt_spec,
            _full((D, D)), _row_spec, _full((1, D)), _full((1, D)),
            _full((1, 1)),
        ],
        out_specs=_row_spec,
        out_shape=jax.ShapeDtypeStruct((N, D), jnp.float32),
    )(a0, a1, c0, c1, WlT, tmp, lnw, lnb, pw)


def kernel(x, edge_index, Wl0, bl0, Wr0, lnw0, lnb0, Wl1, bl1, Wr1, lnw1,
           lnb1, Wl2, bl2, Wr2, lnw2, lnb2, prelu_w):
    ei = edge_index.astype(jnp.int32)
    src = ei[0].reshape(NW, NCH, CH)
    dst = ei[1].reshape(NW, NCH, CH)

    cnt = _sc_count(dst)[:, :N, :16]
    c0, c1 = cnt[0], cnt[1]
    pw = prelu_w.reshape(1, 1)

    params = [
        (Wl0, bl0, Wr0, lnw0, lnb0),
        (Wl1, bl1, Wr1, lnw1, lnb1),
        (Wl2, bl2, Wr2, lnw2, lnb2),
    ]
    h = x
    for i, (Wl, bl, Wr, lnw, lnb) in enumerate(params):
        agg = _sc_segsum(h, src, dst)[:, :N]
        tmp = _tc_pre(h, Wr.T, bl.reshape(1, D), i > 0)
        h = _tc_post(agg[0], agg[1], c0, c1, Wl.T, tmp,
                     lnw.reshape(1, D), lnb.reshape(1, D), pw)
    return h


# speedup vs baseline: 1.0719x; 1.0064x over previous
import functools

import jax
import jax.numpy as jnp
from jax import lax
from jax.experimental import pallas as pl
from jax.experimental.pallas import tpu as pltpu
from jax.experimental.pallas import tpu_sc as plsc

N = 10000
E = 320000
D = 128

NC = 2
NS = 16
NW = NC * NS
EPW = E // NW
CH = 40
NCH = EPW // CH
NP = 10240
RPS = NP // NS
CHC = 80
NCHC = EPW // CHC
ZR = 16

_mesh = plsc.VectorSubcoreMesh(core_axis_name="c", subcore_axis_name="s")


def _zero_2d(ref, rows, cols):
    zero = jnp.zeros((16,), jnp.float32)
    cchunks = cols // 16

    def body(i, carry):
        r = i // cchunks
        j = i % cchunks
        ref[r, pl.ds(j * 16, 16)] = zero
        return carry

    lax.fori_loop(0, rows * cchunks, body, 0)


K = 5
K2 = 2 * K
STEADY = NCH - K2


@functools.partial(
    pl.kernel,
    out_type=jax.ShapeDtypeStruct((NC, NP, D), jnp.float32),
    mesh=_mesh,
    scratch_types=[
        pltpu.VMEM_SHARED((NP, D), jnp.float32),
        pltpu.VMEM((K2, CH), jnp.int32),
        pltpu.VMEM((K2, CH), jnp.int32),
        pltpu.VMEM((K, CH, D), jnp.float32),
        pltpu.VMEM((ZR, D), jnp.float32),
        [pltpu.SemaphoreType.DMA] * K2,
        [pltpu.SemaphoreType.DMA] * K,
        [pltpu.SemaphoreType.DMA] * K,
        pltpu.SemaphoreType.DMA,
    ],
)
def _sc_segsum(h_hbm, src_hbm, dst_hbm, out_hbm, acc_sh, srcr, dstr,
               rows_v, stage_v, semi, semg, sems, semd):
    c = lax.axis_index("c")
    s = lax.axis_index("s")
    wid = s * NC + c
    row0 = s * RPS

    def fire_idx(g, slot):
        pltpu.async_copy(src_hbm.at[wid, g], srcr.at[slot], semi[slot])
        pltpu.async_copy(dst_hbm.at[wid, g], dstr.at[slot], semi[slot])

    def wait_idx(slot):
        pltpu.make_async_copy(src_hbm.at[wid, 0], srcr.at[slot],
                              semi[slot]).wait()
        pltpu.make_async_copy(dst_hbm.at[wid, 0], dstr.at[slot],
                              semi[slot]).wait()

    def fire_gather(slot, rslot):
        pltpu.async_copy(h_hbm.at[srcr.at[slot]], rows_v.at[rslot],
                         semg[rslot])

    def wait_gather(slot, rslot):
        pltpu.make_async_copy(h_hbm.at[srcr.at[slot]], rows_v.at[rslot],
                              semg[rslot]).wait()

    def fire_scatter(slot, rslot):
        pltpu.async_copy(rows_v.at[rslot], acc_sh.at[dstr.at[slot]],
                         sems[rslot], add=True)

    def wait_scatter(slot, rslot):
        pltpu.make_async_copy(rows_v.at[rslot], acc_sh.at[dstr.at[slot]],
                              sems[rslot]).wait()

    for g in range(K2):
        fire_idx(g, g)

    _zero_2d(stage_v, ZR, D)
    for j in range(RPS // ZR):
        pltpu.async_copy(stage_v, acc_sh.at[pl.ds(row0 + j * ZR, ZR)], semd)

    for g in range(K):
        wait_idx(g)
        fire_gather(g, g)

    for j in range(RPS // ZR):
        pltpu.make_async_copy(stage_v, acc_sh.at[pl.ds(row0, ZR)], semd).wait()
    plsc.subcore_barrier()

    def outer(o, carry):
        for b in range(K2):
            g = o * K2 + b
            rb = b % K
            wait_gather(b, rb)
            fire_scatter(b, rb)
            wait_scatter(b, rb)
            fire_idx(g + K2, b)
            wait_idx((b + K) % K2)
            fire_gather((b + K) % K2, rb)
        return carry

    lax.fori_loop(0, STEADY // K2, outer, 0)

    for g in range(STEADY, NCH):
        b = g % K2
        rb = b % K
        wait_gather(b, rb)
        fire_scatter(b, rb)
        wait_scatter(b, rb)
        if g + K < NCH:
            wait_idx((b + K) % K2)
            fire_gather((b + K) % K2, rb)

    plsc.subcore_barrier()

    for j in range(RPS // ZR):
        r = row0 + j * ZR
        pltpu.async_copy(acc_sh.at[pl.ds(r, ZR)], out_hbm.at[c, pl.ds(r, ZR)],
                         semd)
    for j in range(RPS // ZR):
        pltpu.make_async_copy(acc_sh.at[pl.ds(row0, ZR)],
                              out_hbm.at[c, pl.ds(row0, ZR)], semd).wait()


@functools.partial(
    pl.kernel,
    out_type=jax.ShapeDtypeStruct((NC, NP, D), jnp.float32),
    mesh=_mesh,
    scratch_types=[
        pltpu.VMEM_SHARED((NP, D), jnp.float32),
        pltpu.VMEM((NCHC, CHC), jnp.int32),
        pltpu.VMEM((CHC, D), jnp.float32),
        pltpu.VMEM((ZR, D), jnp.float32),
        [pltpu.SemaphoreType.DMA] * K,
        pltpu.SemaphoreType.DMA,
    ],
)
def _sc_count(dst_hbm, out_hbm, cnt_sh, dst_v, ones_v, stage_v, sems, semd):
    c = lax.axis_index("c")
    s = lax.axis_index("s")
    wid = s * NC + c
    row0 = s * RPS

    pltpu.sync_copy(dst_hbm.at[wid], dst_v)

    _zero_2d(stage_v, ZR, D)
    for j in range(RPS // ZR):
        pltpu.async_copy(stage_v, cnt_sh.at[pl.ds(row0 + j * ZR, ZR)], semd)

    one = jnp.ones((16,), jnp.float32)

    def fill(i, carry):
        r = i // 8
        j = i % 8
        ones_v[r, pl.ds(j * 16, 16)] = one
        return carry

    lax.fori_loop(0, CHC * 8, fill, 0)
    for j in range(RPS // ZR):
        pltpu.make_async_copy(stage_v, cnt_sh.at[pl.ds(row0, ZR)], semd).wait()
    plsc.subcore_barrier()

    for b in range(K):
        pltpu.async_copy(ones_v, cnt_sh.at[dst_v.at[b]], sems[b], add=True)

    def outer(o, carry):
        for b in range(K):
            g = o * K + b
            pltpu.make_async_copy(
                ones_v, cnt_sh.at[dst_v.at[g]], sems[b]).wait()

            @pl.when(g + K < NCHC)
            def _():
                pltpu.async_copy(
                    ones_v, cnt_sh.at[dst_v.at[g + K]], sems[b], add=True)
        return carry

    lax.fori_loop(0, NCHC // K, outer, 0)
    plsc.subcore_barrier()

    for j in range(RPS // ZR):
        r = row0 + j * ZR
        pltpu.async_copy(cnt_sh.at[pl.ds(r, ZR)], out_hbm.at[c, pl.ds(r, ZR)],
                         semd)
    for j in range(RPS // ZR):
        pltpu.make_async_copy(cnt_sh.at[pl.ds(row0, ZR)],
                              out_hbm.at[c, pl.ds(row0, ZR)], semd).wait()


_R = 2000
_row_spec = pl.BlockSpec((_R, D), lambda i: (i, 0))
_cnt_spec = pl.BlockSpec((_R, 16), lambda i: (i, 0))


def _full(shape):
    return pl.BlockSpec(shape, lambda i: (0,) * len(shape))


def _tc_pre_body(with_residual, h_ref, wr_ref, bl_ref, out_ref):
    t = jnp.dot(h_ref[...], wr_ref[...], preferred_element_type=jnp.float32,
                precision=lax.Precision.DEFAULT) + bl_ref[...]
    if with_residual:
        t = t + h_ref[...]
    out_ref[...] = t


def _tc_pre(h, WrT, bl, with_residual):
    return pl.pallas_call(
        functools.partial(_tc_pre_body, with_residual),
        grid=(N // _R,),
        in_specs=[_row_spec, _full((D, D)), _full((1, D))],
        out_specs=_row_spec,
        out_shape=jax.ShapeDtypeStruct((N, D), jnp.float32),
    )(h, WrT, bl)


def _tc_post_body(a0_ref, a1_ref, c0_ref, c1_ref, wl_ref, tmp_ref,
                  lnw_ref, lnb_ref, pw_ref, out_ref):
    cnt = c0_ref[:, 0:1] + c1_ref[:, 0:1]
    mean = (a0_ref[...] + a1_ref[...]) / jnp.maximum(cnt, 1.0)
    t = jnp.dot(mean, wl_ref[...], preferred_element_type=jnp.float32,
                precision=lax.Precision.DEFAULT) + tmp_ref[...]
    mu = jnp.mean(t, axis=-1, keepdims=True)
    var = jnp.mean((t - mu) ** 2, axis=-1, keepdims=True)
    t = (t - mu) * lax.rsqrt(var + 1e-5) * lnw_ref[...] + lnb_ref[...]
    w = pw_ref[0, 0]
    out_ref[...] = jnp.where(t > 0, t, w * t)


def _tc_post(a0, a1, c0, c1, WlT, tmp, lnw, lnb, pw):
    return pl.pallas_call(
        _tc_post_body,
        grid=(N // _R,),
        in_specs=[
            _row_spec, _row_spec, _cnt_spec, _cnt_spec,
            _full((D, D)), _row_spec, _full((1, D)), _full((1, D)),
            _full((1, 1)),
        ],
        out_specs=_row_spec,
        out_shape=jax.ShapeDtypeStruct((N, D), jnp.float32),
    )(a0, a1, c0, c1, WlT, tmp, lnw, lnb, pw)


def kernel(x, edge_index, Wl0, bl0, Wr0, lnw0, lnb0, Wl1, bl1, Wr1, lnw1,
           lnb1, Wl2, bl2, Wr2, lnw2, lnb2, prelu_w):
    ei = edge_index.astype(jnp.int32)
    src = ei[0].reshape(NW, NCH, CH)
    dst = ei[1].reshape(NW, NCH, CH)

    dstc = ei[1].reshape(NW, NCHC, CHC)
    cnt = _sc_count(dstc)[:, :N, :16]
    c0, c1 = cnt[0], cnt[1]
    pw = prelu_w.reshape(1, 1)

    params = [
        (Wl0, bl0, Wr0, lnw0, lnb0),
        (Wl1, bl1, Wr1, lnw1, lnb1),
        (Wl2, bl2, Wr2, lnw2, lnb2),
    ]
    h = x
    for i, (Wl, bl, Wr, lnw, lnb) in enumerate(params):
        agg = _sc_segsum(h, src, dst)[:, :N]
        tmp = _tc_pre(h, Wr.T, bl.reshape(1, D), i > 0)
        h = _tc_post(agg[0], agg[1], c0, c1, Wl.T, tmp,
                     lnw.reshape(1, D), lnb.reshape(1, D), pw)
    return h


# speedup vs baseline: 1.1346x; 1.0585x over previous
import functools

import jax
import jax.numpy as jnp
from jax import lax
from jax.experimental import pallas as pl
from jax.experimental.pallas import tpu as pltpu
from jax.experimental.pallas import tpu_sc as plsc

N = 10000
E = 320000
D = 128

NC = 2
NS = 16
NW = NC * NS
EPW = E // NW
CH = 40
NCH = EPW // CH
NP = 10240
RPS = NP // NS
CHC = 80
NCHC = EPW // CHC
ZR = 16

_mesh = plsc.VectorSubcoreMesh(core_axis_name="c", subcore_axis_name="s")


def _zero_2d(ref, rows, cols):
    zero = jnp.zeros((16,), jnp.float32)
    cchunks = cols // 16

    def body(i, carry):
        r = i // cchunks
        j = i % cchunks
        ref[r, pl.ds(j * 16, 16)] = zero
        return carry

    lax.fori_loop(0, rows * cchunks, body, 0)


K = 5
K2 = 2 * K
STEADY = NCH - K2


@functools.partial(
    pl.kernel,
    out_type=jax.ShapeDtypeStruct((NC, NP, D), jnp.float32),
    mesh=_mesh,
    scratch_types=[
        pltpu.VMEM_SHARED((NP, D), jnp.float32),
        pltpu.VMEM((K2, CH), jnp.int32),
        pltpu.VMEM((K2, CH), jnp.int32),
        pltpu.VMEM((K, CH, D), jnp.float32),
        pltpu.VMEM((ZR, D), jnp.float32),
        [pltpu.SemaphoreType.DMA] * K2,
        [pltpu.SemaphoreType.DMA] * K,
        [pltpu.SemaphoreType.DMA] * K,
        pltpu.SemaphoreType.DMA,
    ],
)
def _sc_segsum(h_hbm, src_hbm, dst_hbm, out_hbm, acc_sh, srcr, dstr,
               rows_v, stage_v, semi, semg, sems, semd):
    c = lax.axis_index("c")
    s = lax.axis_index("s")
    wid = s * NC + c
    row0 = s * RPS

    def fire_idx(g, slot):
        pltpu.async_copy(src_hbm.at[wid, g], srcr.at[slot], semi[slot])
        pltpu.async_copy(dst_hbm.at[wid, g], dstr.at[slot], semi[slot])

    def wait_idx(slot):
        pltpu.make_async_copy(src_hbm.at[wid, 0], srcr.at[slot],
                              semi[slot]).wait()
        pltpu.make_async_copy(dst_hbm.at[wid, 0], dstr.at[slot],
                              semi[slot]).wait()

    def fire_gather(slot, rslot):
        pltpu.async_copy(h_hbm.at[srcr.at[slot]], rows_v.at[rslot],
                         semg[rslot])

    def wait_gather(slot, rslot):
        pltpu.make_async_copy(h_hbm.at[srcr.at[slot]], rows_v.at[rslot],
                              semg[rslot]).wait()

    def fire_scatter(slot, rslot):
        pltpu.async_copy(rows_v.at[rslot], acc_sh.at[dstr.at[slot]],
                         sems[rslot], add=True)

    def wait_scatter(slot, rslot):
        pltpu.make_async_copy(rows_v.at[rslot], acc_sh.at[dstr.at[slot]],
                              sems[rslot]).wait()

    for g in range(K2):
        fire_idx(g, g)

    _zero_2d(stage_v, ZR, D)
    for j in range(RPS // ZR):
        pltpu.async_copy(stage_v, acc_sh.at[pl.ds(row0 + j * ZR, ZR)], semd)

    for g in range(K):
        wait_idx(g)
        fire_gather(g, g)

    for j in range(RPS // ZR):
        pltpu.make_async_copy(stage_v, acc_sh.at[pl.ds(row0, ZR)], semd).wait()
    plsc.subcore_barrier()

    def outer(o, carry):
        for b in range(K2):
            g = o * K2 + b
            rb = b % K
            wait_gather(b, rb)
            fire_scatter(b, rb)
            wait_scatter(b, rb)
            fire_idx(g + K2, b)
            wait_idx((b + K) % K2)
            fire_gather((b + K) % K2, rb)
        return carry

    lax.fori_loop(0, STEADY // K2, outer, 0)

    for g in range(STEADY, NCH):
        b = g % K2
        rb = b % K
        wait_gather(b, rb)
        fire_scatter(b, rb)
        wait_scatter(b, rb)
        if g + K < NCH:
            wait_idx((b + K) % K2)
            fire_gather((b + K) % K2, rb)

    plsc.subcore_barrier()

    for j in range(RPS // ZR):
        r = row0 + j * ZR
        pltpu.async_copy(acc_sh.at[pl.ds(r, ZR)], out_hbm.at[c, pl.ds(r, ZR)],
                         semd)
    for j in range(RPS // ZR):
        pltpu.make_async_copy(acc_sh.at[pl.ds(row0, ZR)],
                              out_hbm.at[c, pl.ds(row0, ZR)], semd).wait()


@functools.partial(
    pl.kernel,
    out_type=jax.ShapeDtypeStruct((NC, NP, D), jnp.float32),
    mesh=_mesh,
    scratch_types=[
        pltpu.VMEM_SHARED((NP, D), jnp.float32),
        pltpu.VMEM((NCHC, CHC), jnp.int32),
        pltpu.VMEM((CHC, D), jnp.float32),
        pltpu.VMEM((ZR, D), jnp.float32),
        [pltpu.SemaphoreType.DMA] * K,
        pltpu.SemaphoreType.DMA,
    ],
)
def _sc_count(dst_hbm, out_hbm, cnt_sh, dst_v, ones_v, stage_v, sems, semd):
    c = lax.axis_index("c")
    s = lax.axis_index("s")
    wid = s * NC + c
    row0 = s * RPS

    pltpu.sync_copy(dst_hbm.at[wid], dst_v)

    _zero_2d(stage_v, ZR, D)
    for j in range(RPS // ZR):
        pltpu.async_copy(stage_v, cnt_sh.at[pl.ds(row0 + j * ZR, ZR)], semd)

    one = jnp.ones((16,), jnp.float32)

    def fill(i, carry):
        r = i // 8
        j = i % 8
        ones_v[r, pl.ds(j * 16, 16)] = one
        return carry

    lax.fori_loop(0, CHC * 8, fill, 0)
    for j in range(RPS // ZR):
        pltpu.make_async_copy(stage_v, cnt_sh.at[pl.ds(row0, ZR)], semd).wait()
    plsc.subcore_barrier()

    for b in range(K):
        pltpu.async_copy(ones_v, cnt_sh.at[dst_v.at[b]], sems[b], add=True)

    def outer(o, carry):
        for b in range(K):
            g = o * K + b
            pltpu.make_async_copy(
                ones_v, cnt_sh.at[dst_v.at[g]], sems[b]).wait()

            @pl.when(g + K < NCHC)
            def _():
                pltpu.async_copy(
                    ones_v, cnt_sh.at[dst_v.at[g + K]], sems[b], add=True)
        return carry

    lax.fori_loop(0, NCHC // K, outer, 0)
    plsc.subcore_barrier()

    for j in range(RPS // ZR):
        r = row0 + j * ZR
        pltpu.async_copy(cnt_sh.at[pl.ds(r, ZR)], out_hbm.at[c, pl.ds(r, ZR)],
                         semd)
    for j in range(RPS // ZR):
        pltpu.make_async_copy(cnt_sh.at[pl.ds(row0, ZR)],
                              out_hbm.at[c, pl.ds(row0, ZR)], semd).wait()


_R = 2000
_row_spec = pl.BlockSpec((_R, D), lambda i: (i, 0))
_cnt_spec = pl.BlockSpec((_R, 16), lambda i: (i, 0))


def _full(shape):
    return pl.BlockSpec(shape, lambda i: (0,) * len(shape))


def _tc_pre_body(with_residual, h_ref, wr_ref, bl_ref, out_ref):
    t = jnp.dot(h_ref[...], wr_ref[...], preferred_element_type=jnp.float32,
                precision=lax.Precision.DEFAULT) + bl_ref[...]
    if with_residual:
        t = t + h_ref[...]
    out_ref[...] = t


def _tc_pre(h, WrT, bl, with_residual):
    return pl.pallas_call(
        functools.partial(_tc_pre_body, with_residual),
        grid=(N // _R,),
        in_specs=[_row_spec, _full((D, D)), _full((1, D))],
        out_specs=_row_spec,
        out_shape=jax.ShapeDtypeStruct((N, D), jnp.float32),
    )(h, WrT, bl)


def _tc_post_body(a0_ref, a1_ref, c0_ref, c1_ref, wl_ref, tmp_ref,
                  lnw_ref, lnb_ref, pw_ref, out_ref):
    cnt = c0_ref[0, :, 0:1] + c1_ref[0, :, 0:1]
    mean = (a0_ref[0] + a1_ref[0]) / jnp.maximum(cnt, 1.0)
    t = jnp.dot(mean, wl_ref[...], preferred_element_type=jnp.float32,
                precision=lax.Precision.DEFAULT) + tmp_ref[...]
    mu = jnp.mean(t, axis=-1, keepdims=True)
    var = jnp.mean((t - mu) ** 2, axis=-1, keepdims=True)
    t = (t - mu) * lax.rsqrt(var + 1e-5) * lnw_ref[...] + lnb_ref[...]
    w = pw_ref[0, 0]
    out_ref[...] = jnp.where(t > 0, t, w * t)


def _tc_post(agg, cnt, WlT, tmp, lnw, lnb, pw):
    part0 = pl.BlockSpec((1, _R, D), lambda i: (0, i, 0))
    part1 = pl.BlockSpec((1, _R, D), lambda i: (1, i, 0))
    cnt0 = pl.BlockSpec((1, _R, D), lambda i: (0, i, 0))
    cnt1 = pl.BlockSpec((1, _R, D), lambda i: (1, i, 0))
    return pl.pallas_call(
        _tc_post_body,
        grid=(N // _R,),
        in_specs=[
            part0, part1, cnt0, cnt1,
            _full((D, D)), _row_spec, _full((1, D)), _full((1, D)),
            _full((1, 1)),
        ],
        out_specs=_row_spec,
        out_shape=jax.ShapeDtypeStruct((N, D), jnp.float32),
    )(agg, agg, cnt, cnt, WlT, tmp, lnw, lnb, pw)


def kernel(x, edge_index, Wl0, bl0, Wr0, lnw0, lnb0, Wl1, bl1, Wr1, lnw1,
           lnb1, Wl2, bl2, Wr2, lnw2, lnb2, prelu_w):
    ei = edge_index.astype(jnp.int32)
    src = ei[0].reshape(NW, NCH, CH)
    dst = ei[1].reshape(NW, NCH, CH)

    dstc = ei[1].reshape(NW, NCHC, CHC)
    cnt = _sc_count(dstc)
    pw = prelu_w.reshape(1, 1)

    params = [
        (Wl0, bl0, Wr0, lnw0, lnb0),
        (Wl1, bl1, Wr1, lnw1, lnb1),
        (Wl2, bl2, Wr2, lnw2, lnb2),
    ]
    h = x
    for i, (Wl, bl, Wr, lnw, lnb) in enumerate(params):
        agg = _sc_segsum(h, src, dst)
        tmp = _tc_pre(h, Wr.T, bl.reshape(1, D), i > 0)
        h = _tc_post(agg, cnt, Wl.T, tmp,
                     lnw.reshape(1, D), lnb.reshape(1, D), pw)
    return h
